# Initial kernel scaffold; baseline (speedup 1.0000x reference)
import jax
import jax.numpy as jnp
from jax.experimental import pallas as pl


def kernel(x, W_enc, W_dec, b_enc, b_dec):
    raise NotImplementedError("write your pallas kernel here")


# speedup vs baseline: 11.8111x; 11.8111x over previous
import jax
import jax.numpy as jnp
from jax.experimental import pallas as pl

_K = 64
_EPS = 1e-5
_POS_INF_BITS = 0x7F800000


def _enc_body(x_ref, w_ref, b_ref, pre_ref, mean_ref, std_ref):
    x = x_ref[...]
    mu = jnp.mean(x, axis=1, keepdims=True)
    c = x - mu
    var = jnp.mean(c * c, axis=1, keepdims=True)
    ln = c / jnp.sqrt(var + _EPS)
    pre = jnp.dot(ln, w_ref[...], preferred_element_type=jnp.float32,
                  precision=jax.lax.Precision.DEFAULT)
    pre_ref[...] = pre + b_ref[...]
    mean_ref[...] = mu
    std_ref[...] = jnp.sqrt(var)


def _sel_body(pre_ref, t_ref):
    pre = pre_ref[...]
    rows = pre.shape[0]
    lo0 = jnp.zeros((rows, 1), jnp.int32)
    hi0 = jnp.full((rows, 1), _POS_INF_BITS, jnp.int32)

    def step(_, lohi):
        lo, hi = lohi
        mid = lo + (hi - lo) // 2
        tmid = jax.lax.bitcast_convert_type(mid, jnp.float32)
        cnt = jnp.sum(jnp.where(pre >= tmid, 1.0, 0.0), axis=1, keepdims=True)
        ge = cnt >= _K
        return jnp.where(ge, mid, lo), jnp.where(ge, hi, mid)

    lo, _ = jax.lax.fori_loop(0, 31, step, (lo0, hi0))
    t_ref[...] = jax.lax.bitcast_convert_type(lo, jnp.float32)


def _dec_body(pre_ref, w_ref, t_ref, mean_ref, std_ref, b_ref, out_ref,
              *, num_h):
    h = pl.program_id(1)
    pre = pre_ref[...]
    lat = jnp.where(pre >= t_ref[...], jnp.maximum(pre, 0.0), 0.0)
    contrib = jnp.dot(lat, w_ref[...], preferred_element_type=jnp.float32,
                      precision=jax.lax.Precision.DEFAULT)

    @pl.when(h == 0)
    def _():
        out_ref[...] = contrib

    @pl.when(h > 0)
    def _():
        out_ref[...] = out_ref[...] + contrib

    @pl.when(h == num_h - 1)
    def _():
        out_ref[...] = ((out_ref[...] + b_ref[...]) * std_ref[...]
                        + mean_ref[...])


def kernel(x, W_enc, W_dec, b_enc, b_dec):
    T, D = x.shape
    H = W_enc.shape[1]
    TT = min(1024, T)
    HT = min(2048, H)
    TS = min(128, T)
    nt, nh, ns = T // TT, H // HT, T // TS

    b_enc2 = b_enc.reshape(1, H)
    b_dec2 = b_dec.reshape(1, D)
    f32 = jnp.float32

    pre, mean, std = pl.pallas_call(
        _enc_body,
        grid=(nt, nh),
        in_specs=[
            pl.BlockSpec((TT, D), lambda t, h: (t, 0)),
            pl.BlockSpec((D, HT), lambda t, h: (0, h)),
            pl.BlockSpec((1, HT), lambda t, h: (0, h)),
        ],
        out_specs=[
            pl.BlockSpec((TT, HT), lambda t, h: (t, h)),
            pl.BlockSpec((TT, 1), lambda t, h: (t, 0)),
            pl.BlockSpec((TT, 1), lambda t, h: (t, 0)),
        ],
        out_shape=[
            jax.ShapeDtypeStruct((T, H), f32),
            jax.ShapeDtypeStruct((T, 1), f32),
            jax.ShapeDtypeStruct((T, 1), f32),
        ],
    )(x, W_enc, b_enc2)

    thresh = pl.pallas_call(
        _sel_body,
        grid=(ns,),
        in_specs=[pl.BlockSpec((TS, H), lambda t: (t, 0))],
        out_specs=pl.BlockSpec((TS, 1), lambda t: (t, 0)),
        out_shape=jax.ShapeDtypeStruct((T, 1), f32),
    )(pre)

    import functools
    TD = min(512, T)
    ntd = T // TD
    recon = pl.pallas_call(
        functools.partial(_dec_body, num_h=nh),
        grid=(ntd, nh),
        in_specs=[
            pl.BlockSpec((TD, HT), lambda t, h: (t, h)),
            pl.BlockSpec((HT, D), lambda t, h: (h, 0)),
            pl.BlockSpec((TD, 1), lambda t, h: (t, 0)),
            pl.BlockSpec((TD, 1), lambda t, h: (t, 0)),
            pl.BlockSpec((TD, 1), lambda t, h: (t, 0)),
            pl.BlockSpec((1, D), lambda t, h: (0, 0)),
        ],
        out_specs=pl.BlockSpec((TD, D), lambda t, h: (t, 0)),
        out_shape=jax.ShapeDtypeStruct((T, D), f32),
    )(pre, W_dec, thresh, mean, std, b_dec2)

    return recon
